# baseline (speedup 1.0000x reference)
import functools
import numpy as np
import jax
import jax.numpy as jnp
from jax import lax
from jax.experimental import pallas as pl
from jax.experimental.pallas import tpu as pltpu
from jax.experimental.pallas import tpu_sc as plsc

N = 10000
E = 320000
F = 128
NG = 50
CUTOFF = 5.0
_WIDTH = CUTOFF / (NG - 1)
_COEFF = -0.5 / (_WIDTH * _WIDTH)
_LOG2 = 0.6931471805599453

NC, NS = 2, 16
NW = NC * NS
EPW = E // NW
EC = 40
NECH = EPW // EC
NPAD = 10240
ZPW = NPAD // NW
ZC = 64
NZCH = ZPW // ZC
RC = 40
NRCH = N // RC

_mesh = functools.partial(plsc.VectorSubcoreMesh,
                          core_axis_name="c", subcore_axis_name="s")
_SC_PARAMS = pltpu.CompilerParams(needs_layout_passes=False)


def _prep_body(xh, yh, zzh, a0h, a1h, zh, embh, d2h, s0h,
               xc, yc, zc, a0v, a1v, d2v, zidx, erows, sem):
    cid = lax.axis_index("c")
    sid = lax.axis_index("s")
    wid = sid * NC + cid

    pltpu.sync_copy(xh, xc)
    pltpu.sync_copy(yh, yc)
    pltpu.sync_copy(zzh, zc)
    ebase = wid * EPW
    pltpu.sync_copy(a0h.at[pl.ds(ebase, EPW)], a0v)
    pltpu.sync_copy(a1h.at[pl.ds(ebase, EPW)], a1v)

    def dist_step(i, carry):
        i0 = a0v[pl.ds(i * 16, 16)]
        i1 = a1v[pl.ds(i * 16, 16)]
        dx = plsc.load_gather(xc, [i0]) - plsc.load_gather(xc, [i1])
        dy = plsc.load_gather(yc, [i0]) - plsc.load_gather(yc, [i1])
        dz = plsc.load_gather(zc, [i0]) - plsc.load_gather(zc, [i1])
        d2v[pl.ds(i * 16, 16)] = dx * dx + dy * dy + dz * dz
        return carry

    lax.fori_loop(0, EPW // 16, dist_step, 0)
    pltpu.sync_copy(d2v, d2h.at[pl.ds(ebase, EPW)])

    nbase = wid * ZPW

    def emb_step(k, carry):
        off = nbase + k * ZC
        pltpu.sync_copy(zh.at[pl.ds(off, ZC)], zidx)
        pltpu.async_copy(embh.at[zidx], erows, sem).wait()
        pltpu.sync_copy(erows, s0h.at[pl.ds(off, ZC)])
        return carry

    lax.fori_loop(0, NZCH, emb_step, 0)


@jax.jit
def _prep(xcol, ycol, zcol, a0, a1, zp, embed):
    return pl.kernel(
        _prep_body,
        out_type=(jax.ShapeDtypeStruct((E,), jnp.float32),
                  jax.ShapeDtypeStruct((NPAD, F), jnp.float32)),
        mesh=_mesh(),
        scratch_types=[
            pltpu.VMEM((N,), jnp.float32),
            pltpu.VMEM((N,), jnp.float32),
            pltpu.VMEM((N,), jnp.float32),
            pltpu.VMEM((EPW,), jnp.int32),
            pltpu.VMEM((EPW,), jnp.int32),
            pltpu.VMEM((EPW,), jnp.float32),
            pltpu.VMEM((ZC,), jnp.int32),
            pltpu.VMEM((ZC, F), jnp.float32),
            pltpu.SemaphoreType.DMA,
        ],
        compiler_params=_SC_PARAMS,
    )(xcol, ycol, zcol, a0, a1, zp, embed)


def _msg_body(rnh, wh, a0h, a1h, aggh,
              aggs, a0m, a1m, r00, r01, r10, r11,
              wv0, wv1, semi0, semi1, semg0, semg1, sems0, sems1):
    cid = lax.axis_index("c")
    sid = lax.axis_index("s")
    wid = sid * NC + cid
    r0 = (r00, r01)
    r1 = (r10, r11)
    wv = (wv0, wv1)
    semi = (semi0, semi1)
    semg = (semg0, semg1)
    sems = (sems0, sems1)
    zeros = jnp.zeros((16,), jnp.float32)

    pltpu.sync_copy(a0h.at[pl.ds(wid * EPW, EPW)], a0m)
    pltpu.sync_copy(a1h.at[pl.ds(wid * EPW, EPW)], a1m)

    def idx(m, g):
        return m.at[pl.ds(g * EC, EC)]

    def zrow(i, carry):
        for j in range(F // 16):
            r00[i, pl.ds(j * 16, 16)] = zeros
        return carry

    lax.fori_loop(0, EC, zrow, 0)

    def zchunk(i, carry):
        k = sid + i * NS

        @pl.when(k < NRCH)
        def _():
            pltpu.sync_copy(r00, aggs.at[pl.ds(k * RC, RC)])

        return carry

    lax.fori_loop(0, (NRCH + NS - 1) // NS, zchunk, 0)
    plsc.subcore_barrier()

    ebase = wid * EPW

    def issue_w(g, b):
        pltpu.async_copy(wh.at[pl.ds(ebase + g * EC, EC)], wv[b], semi[b])

    def wait_w(g, b):
        pltpu.make_async_copy(wh.at[pl.ds(ebase + g * EC, EC)], wv[b],
                              semi[b]).wait()

    def issue_g(g, b):
        pltpu.async_copy(rnh.at[idx(a0m, g)], r0[b], semg[b])
        pltpu.async_copy(rnh.at[idx(a1m, g)], r1[b], semg[b])

    def wait_g(g, b):
        pltpu.make_async_copy(rnh.at[idx(a0m, g)], r0[b], semg[b]).wait()
        pltpu.make_async_copy(rnh.at[idx(a1m, g)], r1[b], semg[b]).wait()

    def issue_s(g, b):
        pltpu.async_copy(r0[b], aggs.at[idx(a1m, g)], sems[b], add=True)
        pltpu.async_copy(r1[b], aggs.at[idx(a0m, g)], sems[b], add=True)

    def wait_s(g, b):
        pltpu.make_async_copy(r0[b], aggs.at[idx(a1m, g)], sems[b]).wait()
        pltpu.make_async_copy(r1[b], aggs.at[idx(a0m, g)], sems[b]).wait()

    def compute(b):
        def mul(i, c2):
            for j in range(F // 32):
                wvec = plsc.bitcast(wv[b][i, pl.ds(16 * j, 16)], jnp.bfloat16)
                wlo, whi = plsc.unpack(
                    wvec, format=plsc.PackFormat.INTERLEAVED)
                lo = pl.ds(32 * j, 16)
                hi = pl.ds(32 * j + 16, 16)
                r0[b][i, lo] = r0[b][i, lo] * wlo
                r0[b][i, hi] = r0[b][i, hi] * whi
                r1[b][i, lo] = r1[b][i, lo] * wlo
                r1[b][i, hi] = r1[b][i, hi] * whi
            return c2

        lax.fori_loop(0, EC, mul, 0)

    issue_w(0, 0)
    issue_g(0, 0)
    issue_w(1, 1)

    def outer(k, carry):
        for b in range(2):
            g = 2 * k + b
            nb = 1 - b
            wait_g(g, b)
            wait_w(g, b)

            @pl.when(jnp.logical_and(g + 1 < NECH, g >= 1))
            def _():
                wait_s(g - 1, nb)

            @pl.when(g + 1 < NECH)
            def _():
                issue_g(g + 1, nb)

            compute(b)
            issue_s(g, b)

            @pl.when(g + 2 < NECH)
            def _():
                issue_w(g + 2, b)

        return carry

    lax.fori_loop(0, NECH // 2, outer, 0)
    wait_s(NECH - 2, 0)
    wait_s(NECH - 1, 1)
    plsc.subcore_barrier()

    def ochunk(i, carry):
        k = sid + i * NS

        @pl.when(k < NRCH)
        def _():
            r = k * RC
            pltpu.sync_copy(aggs.at[pl.ds(r, RC)],
                            aggh.at[pl.ds(cid * N + r, RC)])

        return carry

    lax.fori_loop(0, (NRCH + NS - 1) // NS, ochunk, 0)


@jax.jit
def _msg(rn, w, a0, a1):
    return pl.kernel(
        _msg_body,
        out_type=jax.ShapeDtypeStruct((2 * N, F), jnp.float32),
        mesh=_mesh(),
        scratch_types=[
            pltpu.VMEM_SHARED((N, F), jnp.float32),
            pltpu.VMEM((EPW,), jnp.int32),
            pltpu.VMEM((EPW,), jnp.int32),
            pltpu.VMEM((EC, F), jnp.float32),
            pltpu.VMEM((EC, F), jnp.float32),
            pltpu.VMEM((EC, F), jnp.float32),
            pltpu.VMEM((EC, F), jnp.float32),
            pltpu.VMEM((EC, F // 2), jnp.int32),
            pltpu.VMEM((EC, F // 2), jnp.int32),
            pltpu.SemaphoreType.DMA,
            pltpu.SemaphoreType.DMA,
            pltpu.SemaphoreType.DMA,
            pltpu.SemaphoreType.DMA,
            pltpu.SemaphoreType.DMA,
            pltpu.SemaphoreType.DMA,
        ],
        compiler_params=_SC_PARAMS,
    )(rn, w, a0, a1)


def _ssp(x):
    return jax.nn.softplus(x) - _LOG2


def _edge_filter_body(d2_ref, we1, be1, we2, be2, w_ref):
    e = jnp.sqrt(d2_ref[:])
    eb = e.shape[0]
    offs = lax.broadcasted_iota(jnp.int32, (1, NG), 1).astype(jnp.float32) * _WIDTH
    diff = jnp.broadcast_to(e, (eb, NG)) - offs
    g = jnp.exp(_COEFF * diff * diff)
    h = _ssp(jnp.dot(g, we1[:], preferred_element_type=jnp.float32) + be1[:])
    w = jnp.dot(h, we2[:], preferred_element_type=jnp.float32) + be2[:]
    a16 = lax.bitcast_convert_type(w[:, :F // 2].astype(jnp.bfloat16),
                                   jnp.uint16).astype(jnp.uint32)
    b16 = lax.bitcast_convert_type(w[:, F // 2:].astype(jnp.bfloat16),
                                   jnp.uint16).astype(jnp.uint32)
    w_ref[:] = (a16 | (b16 << 16)).astype(jnp.int32)


_EB = 2000


@jax.jit
def _edge_filter(d2, we1, be1, we2, be2):
    return pl.pallas_call(
        _edge_filter_body,
        grid=(E // _EB,),
        in_specs=[
            pl.BlockSpec((_EB, 1), lambda i: (i, 0)),
            pl.BlockSpec((NG, NG), lambda i: (0, 0)),
            pl.BlockSpec((1, NG), lambda i: (0, 0)),
            pl.BlockSpec((NG, F), lambda i: (0, 0)),
            pl.BlockSpec((1, F), lambda i: (0, 0)),
        ],
        out_specs=pl.BlockSpec((_EB, F // 2), lambda i: (i, 0)),
        out_shape=jax.ShapeDtypeStruct((E, F // 2), jnp.int32),
        compiler_params=pltpu.CompilerParams(
            dimension_semantics=("arbitrary",)),
    )(d2, we1, be1, we2, be2)


def _rn_body(s_ref, wn, bn, rn_ref):
    rn_ref[:] = jnp.dot(s_ref[:], wn[:],
                        preferred_element_type=jnp.float32) + bn[:]


_NB = 2000


@jax.jit
def _rn(s, wn, bn):
    return pl.pallas_call(
        _rn_body,
        grid=(N // _NB,),
        in_specs=[
            pl.BlockSpec((_NB, F), lambda i: (i, 0)),
            pl.BlockSpec((F, F), lambda i: (0, 0)),
            pl.BlockSpec((1, F), lambda i: (0, 0)),
        ],
        out_specs=pl.BlockSpec((_NB, F), lambda i: (i, 0)),
        out_shape=jax.ShapeDtypeStruct((N, F), jnp.float32),
        compiler_params=pltpu.CompilerParams(
            dimension_semantics=("arbitrary",)),
    )(s, wn, bn)


def _update_body(a0_ref, a1_ref, s_ref, wu1, bu1, wu2, bu2, out_ref):
    agg = a0_ref[:] + a1_ref[:]
    u = _ssp(jnp.dot(agg, wu1[:], preferred_element_type=jnp.float32) + bu1[:])
    u = jnp.dot(u, wu2[:], preferred_element_type=jnp.float32) + bu2[:]
    out_ref[:] = s_ref[:] + u


@jax.jit
def _update(agg0, agg1, s, wu1, bu1, wu2, bu2):
    return pl.pallas_call(
        _update_body,
        grid=(N // _NB,),
        in_specs=[
            pl.BlockSpec((_NB, F), lambda i: (i, 0)),
            pl.BlockSpec((_NB, F), lambda i: (i, 0)),
            pl.BlockSpec((_NB, F), lambda i: (i, 0)),
            pl.BlockSpec((F, F), lambda i: (0, 0)),
            pl.BlockSpec((1, F), lambda i: (0, 0)),
            pl.BlockSpec((F, F), lambda i: (0, 0)),
            pl.BlockSpec((1, F), lambda i: (0, 0)),
        ],
        out_specs=pl.BlockSpec((_NB, F), lambda i: (i, 0)),
        out_shape=jax.ShapeDtypeStruct((N, F), jnp.float32),
        compiler_params=pltpu.CompilerParams(
            dimension_semantics=("arbitrary",)),
    )(agg0, agg1, s, wu1, bu1, wu2, bu2)


_PERM = np.array([32 * (m // 16) + m % 16 for m in range(64)]
                 + [32 * (m // 16) + 16 + m % 16 for m in range(64)],
                 dtype=np.int32)


def kernel(z, xyz, nbr_list, embed, params):
    a0 = nbr_list[:, 0]
    a1 = nbr_list[:, 1]
    zp = jnp.pad(z, (0, NPAD - N))
    d2, s0p = _prep(xyz[:, 0], xyz[:, 1], xyz[:, 2], a0, a1, zp, embed)
    s = s0p[:N]
    d2 = d2[:, None]
    for p in params:
        w = _edge_filter(d2, p['We1'], p['be1'][None, :],
                         p['We2'][:, _PERM], p['be2'][_PERM][None, :])
        rn = _rn(s, p['Wn'], p['bn'][None, :])
        agg = _msg(rn, w, a0, a1)
        s = _update(agg[:N], agg[N:], s,
                    p['Wu1'], p['bu1'][None, :],
                    p['Wu2'], p['bu2'][None, :])
    return s


# speedup vs baseline: 1.0160x; 1.0160x over previous
import functools
import jax
import jax.numpy as jnp
from jax import lax
from jax.experimental import pallas as pl
from jax.experimental.pallas import tpu as pltpu
from jax.experimental.pallas import tpu_sc as plsc

N = 10000
E = 320000
F = 128
NG = 50
CUTOFF = 5.0
_WIDTH = CUTOFF / (NG - 1)
_COEFF = -0.5 / (_WIDTH * _WIDTH)
_LOG2 = 0.6931471805599453

NC, NS = 2, 16
NW = NC * NS
EPW = E // NW
EC = 40
NECH = EPW // EC
NPAD = 10240
ZPW = NPAD // NW
ZC = 64
NZCH = ZPW // ZC
RC = 40
NRCH = N // RC

_mesh = functools.partial(plsc.VectorSubcoreMesh,
                          core_axis_name="c", subcore_axis_name="s")
_SC_PARAMS = pltpu.CompilerParams(needs_layout_passes=False)


def _prep_body(xh, yh, zzh, a0h, a1h, zh, embh, d2h, s0h,
               xc, yc, zc, a0v, a1v, d2v, zidx, erows, sem):
    cid = lax.axis_index("c")
    sid = lax.axis_index("s")
    wid = sid * NC + cid

    pltpu.sync_copy(xh, xc)
    pltpu.sync_copy(yh, yc)
    pltpu.sync_copy(zzh, zc)
    ebase = wid * EPW
    pltpu.sync_copy(a0h.at[pl.ds(ebase, EPW)], a0v)
    pltpu.sync_copy(a1h.at[pl.ds(ebase, EPW)], a1v)

    def dist_step(i, carry):
        i0 = a0v[pl.ds(i * 16, 16)]
        i1 = a1v[pl.ds(i * 16, 16)]
        dx = plsc.load_gather(xc, [i0]) - plsc.load_gather(xc, [i1])
        dy = plsc.load_gather(yc, [i0]) - plsc.load_gather(yc, [i1])
        dz = plsc.load_gather(zc, [i0]) - plsc.load_gather(zc, [i1])
        d2v[pl.ds(i * 16, 16)] = dx * dx + dy * dy + dz * dz
        return carry

    lax.fori_loop(0, EPW // 16, dist_step, 0)
    pltpu.sync_copy(d2v, d2h.at[pl.ds(ebase, EPW)])

    nbase = wid * ZPW

    def emb_step(k, carry):
        off = nbase + k * ZC
        pltpu.sync_copy(zh.at[pl.ds(off, ZC)], zidx)
        pltpu.async_copy(embh.at[zidx], erows, sem).wait()
        pltpu.sync_copy(erows, s0h.at[pl.ds(off, ZC)])
        return carry

    lax.fori_loop(0, NZCH, emb_step, 0)


@jax.jit
def _prep(xcol, ycol, zcol, a0, a1, zp, embed):
    return pl.kernel(
        _prep_body,
        out_type=(jax.ShapeDtypeStruct((E,), jnp.float32),
                  jax.ShapeDtypeStruct((NPAD, F), jnp.float32)),
        mesh=_mesh(),
        scratch_types=[
            pltpu.VMEM((N,), jnp.float32),
            pltpu.VMEM((N,), jnp.float32),
            pltpu.VMEM((N,), jnp.float32),
            pltpu.VMEM((EPW,), jnp.int32),
            pltpu.VMEM((EPW,), jnp.int32),
            pltpu.VMEM((EPW,), jnp.float32),
            pltpu.VMEM((ZC,), jnp.int32),
            pltpu.VMEM((ZC, F), jnp.float32),
            pltpu.SemaphoreType.DMA,
        ],
        compiler_params=_SC_PARAMS,
    )(xcol, ycol, zcol, a0, a1, zp, embed)


def _msg_body(rnh, wh, a0h, a1h, aggh,
              aggs, a0m, a1m, r00, r01, r10, r11,
              wv0, wv1, semi0, semi1, semg0, semg1, sems0, sems1):
    cid = lax.axis_index("c")
    sid = lax.axis_index("s")
    wid = sid * NC + cid
    r0 = (r00, r01)
    r1 = (r10, r11)
    wv = (wv0, wv1)
    semi = (semi0, semi1)
    semg = (semg0, semg1)
    sems = (sems0, sems1)
    zeros = jnp.zeros((16,), jnp.float32)

    pltpu.sync_copy(a0h.at[pl.ds(wid * EPW, EPW)], a0m)
    pltpu.sync_copy(a1h.at[pl.ds(wid * EPW, EPW)], a1m)

    def idx(m, g):
        return m.at[pl.ds(g * EC, EC)]

    def zrow(i, carry):
        for j in range(F // 16):
            r00[i, pl.ds(j * 16, 16)] = zeros
        return carry

    lax.fori_loop(0, EC, zrow, 0)

    def zchunk(i, carry):
        k = sid + i * NS

        @pl.when(k < NRCH)
        def _():
            pltpu.sync_copy(r00, aggs.at[pl.ds(k * RC, RC)])

        return carry

    lax.fori_loop(0, (NRCH + NS - 1) // NS, zchunk, 0)
    plsc.subcore_barrier()

    ebase = wid * EPW

    def issue_w(g, b):
        pltpu.async_copy(wh.at[pl.ds(ebase + g * EC, EC)], wv[b], semi[b])

    def wait_w(g, b):
        pltpu.make_async_copy(wh.at[pl.ds(ebase + g * EC, EC)], wv[b],
                              semi[b]).wait()

    def issue_g(g, b):
        pltpu.async_copy(rnh.at[idx(a0m, g)], r0[b], semg[b])
        pltpu.async_copy(rnh.at[idx(a1m, g)], r1[b], semg[b])

    def wait_g(g, b):
        pltpu.make_async_copy(rnh.at[idx(a0m, g)], r0[b], semg[b]).wait()
        pltpu.make_async_copy(rnh.at[idx(a1m, g)], r1[b], semg[b]).wait()

    def issue_s(g, b):
        pltpu.async_copy(r0[b], aggs.at[idx(a1m, g)], sems[b], add=True)
        pltpu.async_copy(r1[b], aggs.at[idx(a0m, g)], sems[b], add=True)

    def wait_s(g, b):
        pltpu.make_async_copy(r0[b], aggs.at[idx(a1m, g)], sems[b]).wait()
        pltpu.make_async_copy(r1[b], aggs.at[idx(a0m, g)], sems[b]).wait()

    def compute(b):
        def mul(i, c2):
            for u in range(2):
                e = i * 2 + u
                for j in range(F // 16):
                    sl = pl.ds(j * 16, 16)
                    ww = wv[b][e, sl]
                    r0[b][e, sl] = r0[b][e, sl] * ww
                    r1[b][e, sl] = r1[b][e, sl] * ww
            return c2

        lax.fori_loop(0, EC // 2, mul, 0)

    issue_w(0, 0)
    issue_g(0, 0)
    issue_w(1, 1)

    def outer(k, carry):
        for b in range(2):
            g = 2 * k + b
            nb = 1 - b
            wait_g(g, b)
            wait_w(g, b)

            @pl.when(jnp.logical_and(g + 1 < NECH, g >= 1))
            def _():
                wait_s(g - 1, nb)

            @pl.when(g + 1 < NECH)
            def _():
                issue_g(g + 1, nb)

            compute(b)
            issue_s(g, b)

            @pl.when(g + 2 < NECH)
            def _():
                issue_w(g + 2, b)

        return carry

    lax.fori_loop(0, NECH // 2, outer, 0)
    wait_s(NECH - 2, 0)
    wait_s(NECH - 1, 1)
    plsc.subcore_barrier()

    def ochunk(i, carry):
        k = sid + i * NS

        @pl.when(k < NRCH)
        def _():
            r = k * RC
            pltpu.sync_copy(aggs.at[pl.ds(r, RC)],
                            aggh.at[pl.ds(cid * N + r, RC)])

        return carry

    lax.fori_loop(0, (NRCH + NS - 1) // NS, ochunk, 0)


@jax.jit
def _msg(rn, w, a0, a1):
    return pl.kernel(
        _msg_body,
        out_type=jax.ShapeDtypeStruct((2 * N, F), jnp.float32),
        mesh=_mesh(),
        scratch_types=[
            pltpu.VMEM_SHARED((N, F), jnp.float32),
            pltpu.VMEM((EPW,), jnp.int32),
            pltpu.VMEM((EPW,), jnp.int32),
            pltpu.VMEM((EC, F), jnp.float32),
            pltpu.VMEM((EC, F), jnp.float32),
            pltpu.VMEM((EC, F), jnp.float32),
            pltpu.VMEM((EC, F), jnp.float32),
            pltpu.VMEM((EC, F), jnp.float32),
            pltpu.VMEM((EC, F), jnp.float32),
            pltpu.SemaphoreType.DMA,
            pltpu.SemaphoreType.DMA,
            pltpu.SemaphoreType.DMA,
            pltpu.SemaphoreType.DMA,
            pltpu.SemaphoreType.DMA,
            pltpu.SemaphoreType.DMA,
        ],
        compiler_params=_SC_PARAMS,
    )(rn, w, a0, a1)


def _ssp(x):
    return jax.nn.softplus(x) - _LOG2


def _edge_filter_body(d2_ref, we1, be1, we2, be2, w_ref):
    e = jnp.sqrt(d2_ref[:])
    eb = e.shape[0]
    offs = lax.broadcasted_iota(jnp.int32, (1, NG), 1).astype(jnp.float32) * _WIDTH
    diff = jnp.broadcast_to(e, (eb, NG)) - offs
    g = jnp.exp(_COEFF * diff * diff)
    h = _ssp(jnp.dot(g, we1[:], preferred_element_type=jnp.float32) + be1[:])
    w_ref[:] = jnp.dot(h, we2[:], preferred_element_type=jnp.float32) + be2[:]


_EB = 2000


@jax.jit
def _edge_filter(d2, we1, be1, we2, be2):
    return pl.pallas_call(
        _edge_filter_body,
        grid=(E // _EB,),
        in_specs=[
            pl.BlockSpec((_EB, 1), lambda i: (i, 0)),
            pl.BlockSpec((NG, NG), lambda i: (0, 0)),
            pl.BlockSpec((1, NG), lambda i: (0, 0)),
            pl.BlockSpec((NG, F), lambda i: (0, 0)),
            pl.BlockSpec((1, F), lambda i: (0, 0)),
        ],
        out_specs=pl.BlockSpec((_EB, F), lambda i: (i, 0)),
        out_shape=jax.ShapeDtypeStruct((E, F), jnp.float32),
        compiler_params=pltpu.CompilerParams(
            dimension_semantics=("arbitrary",)),
    )(d2, we1, be1, we2, be2)


def _rn_body(s_ref, wn, bn, rn_ref):
    rn_ref[:] = jnp.dot(s_ref[:], wn[:],
                        preferred_element_type=jnp.float32) + bn[:]


_NB = 2000


@jax.jit
def _rn(s, wn, bn):
    return pl.pallas_call(
        _rn_body,
        grid=(N // _NB,),
        in_specs=[
            pl.BlockSpec((_NB, F), lambda i: (i, 0)),
            pl.BlockSpec((F, F), lambda i: (0, 0)),
            pl.BlockSpec((1, F), lambda i: (0, 0)),
        ],
        out_specs=pl.BlockSpec((_NB, F), lambda i: (i, 0)),
        out_shape=jax.ShapeDtypeStruct((N, F), jnp.float32),
        compiler_params=pltpu.CompilerParams(
            dimension_semantics=("arbitrary",)),
    )(s, wn, bn)


def _update_body(a0_ref, a1_ref, s_ref, wu1, bu1, wu2, bu2, wn, bn,
                 out_ref, rn_ref):
    agg = a0_ref[:] + a1_ref[:]
    u = _ssp(jnp.dot(agg, wu1[:], preferred_element_type=jnp.float32) + bu1[:])
    u = jnp.dot(u, wu2[:], preferred_element_type=jnp.float32) + bu2[:]
    s = s_ref[:] + u
    out_ref[:] = s
    rn_ref[:] = jnp.dot(s, wn[:], preferred_element_type=jnp.float32) + bn[:]


@jax.jit
def _update(agg0, agg1, s, wu1, bu1, wu2, bu2, wn, bn):
    return pl.pallas_call(
        _update_body,
        grid=(N // _NB,),
        in_specs=[
            pl.BlockSpec((_NB, F), lambda i: (i, 0)),
            pl.BlockSpec((_NB, F), lambda i: (i, 0)),
            pl.BlockSpec((_NB, F), lambda i: (i, 0)),
            pl.BlockSpec((F, F), lambda i: (0, 0)),
            pl.BlockSpec((1, F), lambda i: (0, 0)),
            pl.BlockSpec((F, F), lambda i: (0, 0)),
            pl.BlockSpec((1, F), lambda i: (0, 0)),
            pl.BlockSpec((F, F), lambda i: (0, 0)),
            pl.BlockSpec((1, F), lambda i: (0, 0)),
        ],
        out_specs=[
            pl.BlockSpec((_NB, F), lambda i: (i, 0)),
            pl.BlockSpec((_NB, F), lambda i: (i, 0)),
        ],
        out_shape=[
            jax.ShapeDtypeStruct((N, F), jnp.float32),
            jax.ShapeDtypeStruct((N, F), jnp.float32),
        ],
        compiler_params=pltpu.CompilerParams(
            dimension_semantics=("arbitrary",)),
    )(agg0, agg1, s, wu1, bu1, wu2, bu2, wn, bn)


def kernel(z, xyz, nbr_list, embed, params):
    a0 = nbr_list[:, 0]
    a1 = nbr_list[:, 1]
    zp = jnp.pad(z, (0, NPAD - N))
    d2, s0p = _prep(xyz[:, 0], xyz[:, 1], xyz[:, 2], a0, a1, zp, embed)
    s = s0p[:N]
    d2 = d2[:, None]
    ws = [_edge_filter(d2, p['We1'], p['be1'][None, :],
                       p['We2'], p['be2'][None, :]) for p in params]
    rn = _rn(s, params[0]['Wn'], params[0]['bn'][None, :])
    for li, p in enumerate(params):
        agg = _msg(rn, ws[li], a0, a1)
        if li + 1 < len(params):
            pn = params[li + 1]
            s, rn = _update(agg[:N], agg[N:], s,
                            p['Wu1'], p['bu1'][None, :],
                            p['Wu2'], p['bu2'][None, :],
                            pn['Wn'], pn['bn'][None, :])
        else:
            s, _ = _update(agg[:N], agg[N:], s,
                           p['Wu1'], p['bu1'][None, :],
                           p['Wu2'], p['bu2'][None, :],
                           p['Wn'], p['bn'][None, :])
    return s


# speedup vs baseline: 1.0227x; 1.0066x over previous
import functools
import jax
import jax.numpy as jnp
from jax import lax
from jax.experimental import pallas as pl
from jax.experimental.pallas import tpu as pltpu
from jax.experimental.pallas import tpu_sc as plsc

N = 10000
E = 320000
F = 128
NG = 50
CUTOFF = 5.0
_WIDTH = CUTOFF / (NG - 1)
_COEFF = -0.5 / (_WIDTH * _WIDTH)
_LOG2 = 0.6931471805599453

NC, NS = 2, 16
NW = NC * NS
EPW = E // NW
EC = 40
NECH = EPW // EC
NPAD = 10240
ZPW = NPAD // NW
ZC = 64
NZCH = ZPW // ZC
RC = 80
NRCH = N // RC

_mesh = functools.partial(plsc.VectorSubcoreMesh,
                          core_axis_name="c", subcore_axis_name="s")
_SC_PARAMS = pltpu.CompilerParams(needs_layout_passes=False)


def _prep_body(xh, yh, zzh, a0h, a1h, zh, embh, d2h, s0h,
               xc, yc, zc, a0v, a1v, d2v, zidx, erows, sem):
    cid = lax.axis_index("c")
    sid = lax.axis_index("s")
    wid = sid * NC + cid

    pltpu.sync_copy(xh, xc)
    pltpu.sync_copy(yh, yc)
    pltpu.sync_copy(zzh, zc)
    ebase = wid * EPW
    pltpu.sync_copy(a0h.at[pl.ds(ebase, EPW)], a0v)
    pltpu.sync_copy(a1h.at[pl.ds(ebase, EPW)], a1v)

    def dist_step(i, carry):
        i0 = a0v[pl.ds(i * 16, 16)]
        i1 = a1v[pl.ds(i * 16, 16)]
        dx = plsc.load_gather(xc, [i0]) - plsc.load_gather(xc, [i1])
        dy = plsc.load_gather(yc, [i0]) - plsc.load_gather(yc, [i1])
        dz = plsc.load_gather(zc, [i0]) - plsc.load_gather(zc, [i1])
        d2v[pl.ds(i * 16, 16)] = dx * dx + dy * dy + dz * dz
        return carry

    lax.fori_loop(0, EPW // 16, dist_step, 0)
    pltpu.sync_copy(d2v, d2h.at[pl.ds(ebase, EPW)])

    nbase = wid * ZPW

    def emb_step(k, carry):
        off = nbase + k * ZC
        pltpu.sync_copy(zh.at[pl.ds(off, ZC)], zidx)
        pltpu.async_copy(embh.at[zidx], erows, sem).wait()
        pltpu.sync_copy(erows, s0h.at[pl.ds(off, ZC)])
        return carry

    lax.fori_loop(0, NZCH, emb_step, 0)


@jax.jit
def _prep(xcol, ycol, zcol, a0, a1, zp, embed):
    return pl.kernel(
        _prep_body,
        out_type=(jax.ShapeDtypeStruct((E,), jnp.float32),
                  jax.ShapeDtypeStruct((NPAD, F), jnp.float32)),
        mesh=_mesh(),
        scratch_types=[
            pltpu.VMEM((N,), jnp.float32),
            pltpu.VMEM((N,), jnp.float32),
            pltpu.VMEM((N,), jnp.float32),
            pltpu.VMEM((EPW,), jnp.int32),
            pltpu.VMEM((EPW,), jnp.int32),
            pltpu.VMEM((EPW,), jnp.float32),
            pltpu.VMEM((ZC,), jnp.int32),
            pltpu.VMEM((ZC, F), jnp.float32),
            pltpu.SemaphoreType.DMA,
        ],
        compiler_params=_SC_PARAMS,
    )(xcol, ycol, zcol, a0, a1, zp, embed)


def _msg_body(rnh, wh, asch, aggh,
              aggs, asm, rb0, rb1,
              wv0, wv1, semi0, semi1, semg0, semg1, sems0, sems1):
    cid = lax.axis_index("c")
    sid = lax.axis_index("s")
    wid = sid * NC + cid
    rb = (rb0, rb1)
    wv = (wv0, wv1)
    semi = (semi0, semi1)
    semg = (semg0, semg1)
    sems = (sems0, sems1)
    zeros = jnp.zeros((16,), jnp.float32)

    pltpu.sync_copy(asch.at[pl.ds(wid * 2 * EPW, 2 * EPW)], asm)

    def gidx0(g):
        return asm.at[pl.ds(g * 2 * EC + EC, EC)]

    def gidx1(g):
        return asm.at[pl.ds(g * 2 * EC, EC)]

    def sidx(g):
        return asm.at[pl.ds(g * 2 * EC, 2 * EC)]

    def zrow(i, carry):
        for j in range(F // 16):
            rb0[i, pl.ds(j * 16, 16)] = zeros
        return carry

    lax.fori_loop(0, RC, zrow, 0)

    def zchunk(i, carry):
        k = sid + i * NS

        @pl.when(k < NRCH)
        def _():
            pltpu.sync_copy(rb0, aggs.at[pl.ds(k * RC, RC)])

        return carry

    lax.fori_loop(0, (NRCH + NS - 1) // NS, zchunk, 0)
    plsc.subcore_barrier()

    ebase = wid * EPW

    def issue_w(g, b):
        pltpu.async_copy(wh.at[pl.ds(ebase + g * EC, EC)], wv[b], semi[b])

    def wait_w(g, b):
        pltpu.make_async_copy(wh.at[pl.ds(ebase + g * EC, EC)], wv[b],
                              semi[b]).wait()

    def issue_g(g, b):
        pltpu.async_copy(rnh.at[gidx0(g)], rb[b].at[pl.ds(0, EC)], semg[b])
        pltpu.async_copy(rnh.at[gidx1(g)], rb[b].at[pl.ds(EC, EC)], semg[b])

    def wait_g(g, b):
        pltpu.make_async_copy(rnh.at[gidx0(g)], rb[b].at[pl.ds(0, EC)],
                              semg[b]).wait()
        pltpu.make_async_copy(rnh.at[gidx1(g)], rb[b].at[pl.ds(EC, EC)],
                              semg[b]).wait()

    def issue_s(g, b):
        pltpu.async_copy(rb[b], aggs.at[sidx(g)], sems[b], add=True)

    def wait_s(g, b):
        pltpu.make_async_copy(rb[b], aggs.at[sidx(g)], sems[b]).wait()

    def compute(b):
        def mul(i, c2):
            for j in range(F // 16):
                sl = pl.ds(j * 16, 16)
                ww = wv[b][i, sl]
                rb[b][i, sl] = rb[b][i, sl] * ww
                rb[b][EC + i, sl] = rb[b][EC + i, sl] * ww
            return c2

        lax.fori_loop(0, EC, mul, 0)

    issue_w(0, 0)
    issue_g(0, 0)
    issue_w(1, 1)

    def outer(k, carry):
        for b in range(2):
            g = 2 * k + b
            nb = 1 - b
            wait_g(g, b)
            wait_w(g, b)

            @pl.when(jnp.logical_and(g + 1 < NECH, g >= 1))
            def _():
                wait_s(g - 1, nb)

            @pl.when(g + 1 < NECH)
            def _():
                issue_g(g + 1, nb)

            compute(b)
            issue_s(g, b)

            @pl.when(g + 2 < NECH)
            def _():
                issue_w(g + 2, b)

        return carry

    lax.fori_loop(0, NECH // 2, outer, 0)
    wait_s(NECH - 2, 0)
    wait_s(NECH - 1, 1)
    plsc.subcore_barrier()

    def ochunk(i, carry):
        k = sid + i * NS

        @pl.when(k < NRCH)
        def _():
            r = k * RC
            pltpu.sync_copy(aggs.at[pl.ds(r, RC)],
                            aggh.at[pl.ds(cid * N + r, RC)])

        return carry

    lax.fori_loop(0, (NRCH + NS - 1) // NS, ochunk, 0)


@jax.jit
def _msg(rn, w, asc):
    return pl.kernel(
        _msg_body,
        out_type=jax.ShapeDtypeStruct((2 * N, F), jnp.float32),
        mesh=_mesh(),
        scratch_types=[
            pltpu.VMEM_SHARED((N, F), jnp.float32),
            pltpu.VMEM((2 * EPW,), jnp.int32),
            pltpu.VMEM((2 * EC, F), jnp.float32),
            pltpu.VMEM((2 * EC, F), jnp.float32),
            pltpu.VMEM((EC, F), jnp.float32),
            pltpu.VMEM((EC, F), jnp.float32),
            pltpu.SemaphoreType.DMA,
            pltpu.SemaphoreType.DMA,
            pltpu.SemaphoreType.DMA,
            pltpu.SemaphoreType.DMA,
            pltpu.SemaphoreType.DMA,
            pltpu.SemaphoreType.DMA,
        ],
        compiler_params=_SC_PARAMS,
    )(rn, w, asc)


def _ssp(x):
    return jax.nn.softplus(x) - _LOG2


def _edge_filter_body(d2_ref, we1, be1, we2, be2, w_ref):
    e = jnp.sqrt(d2_ref[:])
    eb = e.shape[0]
    offs = lax.broadcasted_iota(jnp.int32, (1, NG), 1).astype(jnp.float32) * _WIDTH
    diff = jnp.broadcast_to(e, (eb, NG)) - offs
    g = jnp.exp(_COEFF * diff * diff)
    h = _ssp(jnp.dot(g, we1[:], preferred_element_type=jnp.float32) + be1[:])
    w_ref[:] = jnp.dot(h, we2[:], preferred_element_type=jnp.float32) + be2[:]


_EB = 2000


@jax.jit
def _edge_filter(d2, we1, be1, we2, be2):
    return pl.pallas_call(
        _edge_filter_body,
        grid=(E // _EB,),
        in_specs=[
            pl.BlockSpec((_EB, 1), lambda i: (i, 0)),
            pl.BlockSpec((NG, NG), lambda i: (0, 0)),
            pl.BlockSpec((1, NG), lambda i: (0, 0)),
            pl.BlockSpec((NG, F), lambda i: (0, 0)),
            pl.BlockSpec((1, F), lambda i: (0, 0)),
        ],
        out_specs=pl.BlockSpec((_EB, F), lambda i: (i, 0)),
        out_shape=jax.ShapeDtypeStruct((E, F), jnp.float32),
        compiler_params=pltpu.CompilerParams(
            dimension_semantics=("arbitrary",)),
    )(d2, we1, be1, we2, be2)


def _rn_body(s_ref, wn, bn, rn_ref):
    rn_ref[:] = jnp.dot(s_ref[:], wn[:],
                        preferred_element_type=jnp.float32) + bn[:]


_NB = 2000


@jax.jit
def _rn(s, wn, bn):
    return pl.pallas_call(
        _rn_body,
        grid=(N // _NB,),
        in_specs=[
            pl.BlockSpec((_NB, F), lambda i: (i, 0)),
            pl.BlockSpec((F, F), lambda i: (0, 0)),
            pl.BlockSpec((1, F), lambda i: (0, 0)),
        ],
        out_specs=pl.BlockSpec((_NB, F), lambda i: (i, 0)),
        out_shape=jax.ShapeDtypeStruct((N, F), jnp.float32),
        compiler_params=pltpu.CompilerParams(
            dimension_semantics=("arbitrary",)),
    )(s, wn, bn)


def _update_body(a0_ref, a1_ref, s_ref, wu1, bu1, wu2, bu2, wn, bn,
                 out_ref, rn_ref):
    agg = a0_ref[:] + a1_ref[:]
    u = _ssp(jnp.dot(agg, wu1[:], preferred_element_type=jnp.float32) + bu1[:])
    u = jnp.dot(u, wu2[:], preferred_element_type=jnp.float32) + bu2[:]
    s = s_ref[:] + u
    out_ref[:] = s
    rn_ref[:] = jnp.dot(s, wn[:], preferred_element_type=jnp.float32) + bn[:]


@jax.jit
def _update(agg0, agg1, s, wu1, bu1, wu2, bu2, wn, bn):
    return pl.pallas_call(
        _update_body,
        grid=(N // _NB,),
        in_specs=[
            pl.BlockSpec((_NB, F), lambda i: (i, 0)),
            pl.BlockSpec((_NB, F), lambda i: (i, 0)),
            pl.BlockSpec((_NB, F), lambda i: (i, 0)),
            pl.BlockSpec((F, F), lambda i: (0, 0)),
            pl.BlockSpec((1, F), lambda i: (0, 0)),
            pl.BlockSpec((F, F), lambda i: (0, 0)),
            pl.BlockSpec((1, F), lambda i: (0, 0)),
            pl.BlockSpec((F, F), lambda i: (0, 0)),
            pl.BlockSpec((1, F), lambda i: (0, 0)),
        ],
        out_specs=[
            pl.BlockSpec((_NB, F), lambda i: (i, 0)),
            pl.BlockSpec((_NB, F), lambda i: (i, 0)),
        ],
        out_shape=[
            jax.ShapeDtypeStruct((N, F), jnp.float32),
            jax.ShapeDtypeStruct((N, F), jnp.float32),
        ],
        compiler_params=pltpu.CompilerParams(
            dimension_semantics=("arbitrary",)),
    )(agg0, agg1, s, wu1, bu1, wu2, bu2, wn, bn)


def kernel(z, xyz, nbr_list, embed, params):
    a0 = nbr_list[:, 0]
    a1 = nbr_list[:, 1]
    asc = jnp.stack((a1.reshape(NW, NECH, EC), a0.reshape(NW, NECH, EC)),
                    axis=2).reshape(-1)
    zp = jnp.pad(z, (0, NPAD - N))
    d2, s0p = _prep(xyz[:, 0], xyz[:, 1], xyz[:, 2], a0, a1, zp, embed)
    s = s0p[:N]
    d2 = d2[:, None]
    ws = [_edge_filter(d2, p['We1'], p['be1'][None, :],
                       p['We2'], p['be2'][None, :]) for p in params]
    rn = _rn(s, params[0]['Wn'], params[0]['bn'][None, :])
    for li, p in enumerate(params):
        agg = _msg(rn, ws[li], asc)
        if li + 1 < len(params):
            pn = params[li + 1]
            s, rn = _update(agg[:N], agg[N:], s,
                            p['Wu1'], p['bu1'][None, :],
                            p['Wu2'], p['bu2'][None, :],
                            pn['Wn'], pn['bn'][None, :])
        else:
            s, _ = _update(agg[:N], agg[N:], s,
                           p['Wu1'], p['bu1'][None, :],
                           p['Wu2'], p['bu2'][None, :],
                           p['Wn'], p['bn'][None, :])
    return s
